# Initial kernel scaffold; baseline (speedup 1.0000x reference)
import jax
import jax.numpy as jnp
from jax.experimental import pallas as pl


def kernel(users, items, user_w, item_w, topic_w, A_indices, A_values):
    raise NotImplementedError("write your pallas kernel here")


# speedup vs baseline: 3.5959x; 3.5959x over previous
import functools

import jax
import jax.numpy as jnp
from jax import lax
from jax.experimental import pallas as pl
from jax.experimental.pallas import tpu as pltpu
from jax.experimental.pallas import tpu_sc as plsc

N_USERS = 20000
N_ITEMS = 29900
N_TOPICS = 100
DIM = 64
LAYERS = 3
NNZ = 800000
N_TOTAL = N_USERS + N_ITEMS + N_TOPICS
BATCH = 4096

NC, NS, L = 2, 16, 16
HALF = N_TOTAL // NC
DUMMY = HALF
C = 128
BLK = 8
MACRO = 25
CHUNKS_PER_TILE = 2 * BLK * MACRO
PAD_E = NS * CHUNKS_PER_TILE * C
FLUSH = 1568
SH_ROWS = NS * FLUSH


def _process_chunk(rows, sidx, rowblk, valblk, kk, sc_base, shared):
    @pl.loop(0, 8)
    def _idx(g):
        r = rowblk[kk, pl.ds(g * L, L)]
        loc = r - sc_base
        oob = (loc < 0) | (loc >= HALF)
        dummy = DUMMY + ((lax.iota(jnp.int32, L) + g * L) & 63)
        sidx[pl.ds(g * L, L)] = jnp.where(oob, dummy, loc)

    @pl.loop(0, 8)
    def _scale(g):
        v16 = valblk[kk, pl.ds(g * L, L)]
        for k in range(16):
            e = g * 16 + k
            bv = jnp.broadcast_to(v16[k], (L,))
            rs = [rows[e, pl.ds(q * L, L)] for q in range(4)]
            for q in range(4):
                rows[e, pl.ds(q * L, L)] = rs[q] * bv

    pltpu.sync_copy(rows, shared.at[sidx], add=True)


def _load_block(col2, row2, val2, cb, rb, vb, sem, goff):
    return (pltpu.async_copy(col2.at[pl.ds(goff, BLK)], cb, sem),
            pltpu.async_copy(row2.at[pl.ds(goff, BLK)], rb, sem),
            pltpu.async_copy(val2.at[pl.ds(goff, BLK)], vb, sem))


def _layer_body(src, row2, col2, val2, out, colA, rowA, valA, colB, rowB,
                valB, sidx, rows0, rows1, shared, gsem0, gsem1, isemA,
                isemB):
    c = lax.axis_index("c")
    s = lax.axis_index("s")
    sc_base = c * HALF
    rows_ = (rows0, rows1)
    gsem_ = (gsem0, gsem1)

    @pl.loop(0, C)
    def _zero(r):
        for q in range(4):
            rows0[r, pl.ds(q * L, L)] = jnp.zeros((L,), jnp.float32)

    zbase = s * FLUSH

    @pl.loop(0, 12)
    def _zcopy(j):
        pltpu.sync_copy(rows0, shared.at[pl.ds(zbase + j * C, C)])

    pltpu.sync_copy(rows0.at[pl.ds(0, FLUSH - 12 * C)],
                    shared.at[pl.ds(zbase + 12 * C, FLUSH - 12 * C)])
    plsc.subcore_barrier()

    tile_chunk0 = s * CHUNKS_PER_TILE

    for d in _load_block(col2, row2, val2, colA, rowA, valA, isemA,
                         tile_chunk0):
        d.wait()
    pltpu.async_copy(src.at[colA.at[0]], rows0, gsem0)

    @pl.loop(0, MACRO)
    def _macro(m):
        base = tile_chunk0 + m * 2 * BLK
        dB = _load_block(col2, row2, val2, colB, rowB, valB, isemB,
                         base + BLK)
        for k in range(16):
            par = k % 2
            if k < 8:
                blk, kk = (colA, rowA, valA), k
            else:
                blk, kk = (colB, rowB, valB), k - 8
            colblk, rowblk, valblk = blk
            if k == 7:
                for d in dB:
                    d.wait()
            if k == 8:
                @pl.when(m < MACRO - 1)
                def _():
                    _load_block(col2, row2, val2, colA, rowA, valA,
                                isemA, base + 2 * BLK)
            if k < 7:
                pltpu.async_copy(src.at[colA.at[k + 1]], rows_[1 - par],
                                 gsem_[1 - par])
            elif k < 15:
                pltpu.async_copy(src.at[colB.at[k - 7]], rows_[1 - par],
                                 gsem_[1 - par])
            else:
                @pl.when(m < MACRO - 1)
                def _():
                    for cb, rb, vb in ((colA, rowA, valA),):
                        (pltpu.make_async_copy(
                            col2.at[pl.ds(tile_chunk0, BLK)], cb, isemA)
                         .wait())
                        (pltpu.make_async_copy(
                            row2.at[pl.ds(tile_chunk0, BLK)], rb, isemA)
                         .wait())
                        (pltpu.make_async_copy(
                            val2.at[pl.ds(tile_chunk0, BLK)], vb, isemA)
                         .wait())
                    pltpu.async_copy(src.at[colA.at[0]], rows0, gsem0)
            pltpu.make_async_copy(src.at[colblk.at[kk]], rows_[par],
                                  gsem_[par]).wait()
            _process_chunk(rows_[par], sidx, rowblk, valblk, kk, sc_base,
                           shared)

    plsc.subcore_barrier()

    fbase = s * FLUSH

    @pl.when(s < NS - 1)
    def _():
        pltpu.sync_copy(shared.at[pl.ds(fbase, FLUSH)],
                        out.at[pl.ds(sc_base + fbase, FLUSH)])

    @pl.when(s == NS - 1)
    def _():
        pltpu.sync_copy(shared.at[pl.ds(fbase, HALF - (NS - 1) * FLUSH)],
                        out.at[pl.ds(sc_base + fbase,
                                     HALF - (NS - 1) * FLUSH)])


_layer = pl.kernel(
    _layer_body,
    out_type=jax.ShapeDtypeStruct((N_TOTAL, DIM), jnp.float32),
    mesh=plsc.VectorSubcoreMesh(core_axis_name="c", subcore_axis_name="s",
                                num_cores=NC, num_subcores=NS),
    scratch_types=[
        pltpu.VMEM((BLK, C), jnp.int32),
        pltpu.VMEM((BLK, C), jnp.int32),
        pltpu.VMEM((BLK, C), jnp.float32),
        pltpu.VMEM((BLK, C), jnp.int32),
        pltpu.VMEM((BLK, C), jnp.int32),
        pltpu.VMEM((BLK, C), jnp.float32),
        pltpu.VMEM((C,), jnp.int32),
        pltpu.VMEM((C, DIM), jnp.float32),
        pltpu.VMEM((C, DIM), jnp.float32),
        pltpu.VMEM_SHARED((SH_ROWS, DIM), jnp.float32),
        pltpu.SemaphoreType.DMA,
        pltpu.SemaphoreType.DMA,
        pltpu.SemaphoreType.DMA,
        pltpu.SemaphoreType.DMA,
    ],
    compiler_params=pltpu.CompilerParams(use_tc_tiling_on_sc=False),
)

B_PER_TILE = BATCH // (NC * NS)


def _gather_body(e0, e1, e2, e3, uidx1, iidx1, usum, isum,
                 idx, acc, tmp, sem):
    c = lax.axis_index("c")
    s = lax.axis_index("s")
    wid = s * NC + c
    base = wid * B_PER_TILE
    tables = (e0, e1, e2, e3)

    for idx1, out in ((uidx1, usum), (iidx1, isum)):
        pltpu.sync_copy(idx1.at[pl.ds(base, B_PER_TILE)], idx)
        pltpu.async_copy(tables[0].at[idx], acc, sem).wait()
        for tbl in tables[1:]:
            pltpu.async_copy(tbl.at[idx], tmp, sem).wait()

            @pl.loop(0, B_PER_TILE)
            def _add(r):
                for q in range(4):
                    acc[r, pl.ds(q * L, L)] = (acc[r, pl.ds(q * L, L)] +
                                               tmp[r, pl.ds(q * L, L)])

        pltpu.sync_copy(acc, out.at[pl.ds(base, B_PER_TILE)])


_gather4 = pl.kernel(
    _gather_body,
    out_type=(jax.ShapeDtypeStruct((BATCH, DIM), jnp.float32),
              jax.ShapeDtypeStruct((BATCH, DIM), jnp.float32)),
    mesh=plsc.VectorSubcoreMesh(core_axis_name="c", subcore_axis_name="s",
                                num_cores=NC, num_subcores=NS),
    scratch_types=[
        pltpu.VMEM((B_PER_TILE,), jnp.int32),
        pltpu.VMEM((B_PER_TILE, DIM), jnp.float32),
        pltpu.VMEM((B_PER_TILE, DIM), jnp.float32),
        pltpu.SemaphoreType.DMA,
    ],
    compiler_params=pltpu.CompilerParams(use_tc_tiling_on_sc=False),
)


def _dot_body(u_ref, i_ref, o_ref):
    o_ref[...] = jnp.sum(u_ref[...] * i_ref[...], axis=1) * (1.0 / 16.0)


def kernel(users, items, user_w, item_w, topic_w, A_indices, A_values):
    all_emb = jnp.concatenate([user_w, item_w, topic_w], axis=0)
    row = A_indices[0]
    col = A_indices[1]
    pad = PAD_E - NNZ
    row2 = jnp.concatenate(
        [row, jnp.full((pad,), 2 ** 30, jnp.int32)]).reshape(-1, C)
    col2 = jnp.concatenate(
        [col, jnp.zeros((pad,), jnp.int32)]).reshape(-1, C)
    val2 = jnp.concatenate(
        [A_values, jnp.zeros((pad,), jnp.float32)]).reshape(-1, C)

    e0 = all_emb
    e1 = _layer(e0, row2, col2, val2)
    e2 = _layer(e1, row2, col2, val2)
    e3 = _layer(e2, row2, col2, val2)

    iidx1 = items + N_USERS
    usum, isum = _gather4(e0, e1, e2, e3, users, iidx1)

    scores = pl.pallas_call(
        _dot_body,
        out_shape=jax.ShapeDtypeStruct((BATCH,), jnp.float32),
    )(usum, isum)
    return scores


# speedup vs baseline: 5.1337x; 1.4277x over previous
import functools

import jax
import jax.numpy as jnp
from jax import lax
from jax.experimental import pallas as pl
from jax.experimental.pallas import tpu as pltpu
from jax.experimental.pallas import tpu_sc as plsc

N_USERS = 20000
N_ITEMS = 29900
N_TOPICS = 100
DIM = 64
LAYERS = 3
NNZ = 800000
N_TOTAL = N_USERS + N_ITEMS + N_TOPICS
BATCH = 4096

NC, NS, L = 2, 16, 16
HALF = N_TOTAL // NC
DUMMY = HALF
C = 128
BLK = 6
DEPTH = 3
MACRO = 33
CHUNKS_PER_TILE = 2 * BLK * MACRO
PAD_E = NS * CHUNKS_PER_TILE * C
FLUSH = 1568
SH_ROWS = NS * FLUSH


def _process_chunk(rows, sidx, rowblk, valblk, kk, sc_base, shared):
    @pl.loop(0, 8)
    def _idx(g):
        r = rowblk[kk, pl.ds(g * L, L)]
        loc = r - sc_base
        oob = (loc < 0) | (loc >= HALF)
        dummy = DUMMY + ((lax.iota(jnp.int32, L) + g * L) & 63)
        sidx[pl.ds(g * L, L)] = jnp.where(oob, dummy, loc)

    @pl.loop(0, 8)
    def _scale(g):
        v16 = valblk[kk, pl.ds(g * L, L)]
        for k in range(16):
            e = g * 16 + k
            bv = jnp.broadcast_to(v16[k], (L,))
            rs = [rows[e, pl.ds(q * L, L)] for q in range(4)]
            for q in range(4):
                rows[e, pl.ds(q * L, L)] = rs[q] * bv

    pltpu.sync_copy(rows, shared.at[sidx], add=True)


def _load_block(col2, row2, val2, cb, rb, vb, sem, goff):
    return (pltpu.async_copy(col2.at[pl.ds(goff, BLK)], cb, sem),
            pltpu.async_copy(row2.at[pl.ds(goff, BLK)], rb, sem),
            pltpu.async_copy(val2.at[pl.ds(goff, BLK)], vb, sem))


def _layer_body(src, row2, col2, val2, out, colA, rowA, valA, colB, rowB,
                valB, sidx, rows0, rows1, rows2, shared, gsem0,
                gsem1, gsem2, isemA, isemB):
    c = lax.axis_index("c")
    s = lax.axis_index("s")
    sc_base = c * HALF
    rows_ = (rows0, rows1, rows2)
    gsem_ = (gsem0, gsem1, gsem2)

    @pl.loop(0, C)
    def _zero(r):
        for q in range(4):
            rows0[r, pl.ds(q * L, L)] = jnp.zeros((L,), jnp.float32)

    zbase = s * FLUSH

    @pl.loop(0, 12)
    def _zcopy(j):
        pltpu.sync_copy(rows0, shared.at[pl.ds(zbase + j * C, C)])

    pltpu.sync_copy(rows0.at[pl.ds(0, FLUSH - 12 * C)],
                    shared.at[pl.ds(zbase + 12 * C, FLUSH - 12 * C)])
    plsc.subcore_barrier()

    tile_chunk0 = s * CHUNKS_PER_TILE

    def _drain_idx(cb, rb, vb, sem):
        pltpu.make_async_copy(col2.at[pl.ds(tile_chunk0, BLK)], cb,
                              sem).wait()
        pltpu.make_async_copy(row2.at[pl.ds(tile_chunk0, BLK)], rb,
                              sem).wait()
        pltpu.make_async_copy(val2.at[pl.ds(tile_chunk0, BLK)], vb,
                              sem).wait()

    for d in _load_block(col2, row2, val2, colA, rowA, valA, isemA,
                         tile_chunk0):
        d.wait()
    for t in range(DEPTH - 1):
        pltpu.async_copy(src.at[colA.at[t]], rows_[t], gsem_[t])

    @pl.loop(0, MACRO)
    def _macro(m):
        base = tile_chunk0 + m * 2 * BLK
        dB = _load_block(col2, row2, val2, colB, rowB, valB, isemB,
                         base + BLK)
        for k in range(2 * BLK):
            par = k % DEPTH
            if k < BLK:
                rowblk, valblk, kk = rowA, valA, k
            else:
                rowblk, valblk, kk = rowB, valB, k - BLK
            if k == BLK - 3:
                for d in dB:
                    d.wait()
            if k == BLK:
                @pl.when(m < MACRO - 1)
                def _():
                    _load_block(col2, row2, val2, colA, rowA, valA,
                                isemA, base + 2 * BLK)
            npar = (k + 2) % DEPTH
            t = k + 2
            if t < BLK:
                pltpu.async_copy(src.at[colA.at[t]], rows_[npar],
                                 gsem_[npar])
            elif t < 2 * BLK:
                pltpu.async_copy(src.at[colB.at[t - BLK]], rows_[npar],
                                 gsem_[npar])
            else:
                if t == 2 * BLK:
                    @pl.when(m < MACRO - 1)
                    def _():
                        _drain_idx(colA, rowA, valA, isemA)

                @pl.when(m < MACRO - 1)
                def _():
                    pltpu.async_copy(src.at[colA.at[t - 2 * BLK]],
                                     rows_[npar], gsem_[npar])
            pltpu.make_async_copy(src.at[colA.at[0]], rows_[par],
                                  gsem_[par]).wait()
            _process_chunk(rows_[par], sidx, rowblk, valblk, kk, sc_base,
                           shared)

    plsc.subcore_barrier()

    fbase = s * FLUSH

    @pl.when(s < NS - 1)
    def _():
        pltpu.sync_copy(shared.at[pl.ds(fbase, FLUSH)],
                        out.at[pl.ds(sc_base + fbase, FLUSH)])

    @pl.when(s == NS - 1)
    def _():
        pltpu.sync_copy(shared.at[pl.ds(fbase, HALF - (NS - 1) * FLUSH)],
                        out.at[pl.ds(sc_base + fbase,
                                     HALF - (NS - 1) * FLUSH)])


_layer = pl.kernel(
    _layer_body,
    out_type=jax.ShapeDtypeStruct((N_TOTAL, DIM), jnp.float32),
    mesh=plsc.VectorSubcoreMesh(core_axis_name="c", subcore_axis_name="s",
                                num_cores=NC, num_subcores=NS),
    scratch_types=[
        pltpu.VMEM((BLK, C), jnp.int32),
        pltpu.VMEM((BLK, C), jnp.int32),
        pltpu.VMEM((BLK, C), jnp.float32),
        pltpu.VMEM((BLK, C), jnp.int32),
        pltpu.VMEM((BLK, C), jnp.int32),
        pltpu.VMEM((BLK, C), jnp.float32),
        pltpu.VMEM((C,), jnp.int32),
        pltpu.VMEM((C, DIM), jnp.float32),
        pltpu.VMEM((C, DIM), jnp.float32),
        pltpu.VMEM((C, DIM), jnp.float32),
        pltpu.VMEM_SHARED((SH_ROWS, DIM), jnp.float32),
        pltpu.SemaphoreType.DMA,
        pltpu.SemaphoreType.DMA,
        pltpu.SemaphoreType.DMA,
        pltpu.SemaphoreType.DMA,
        pltpu.SemaphoreType.DMA,
    ],
    compiler_params=pltpu.CompilerParams(use_tc_tiling_on_sc=False),
)

B_PER_TILE = BATCH // (NC * NS)


def _gather_body(e0, e1, e2, e3, uidx1, iidx1, usum, isum,
                 idx, acc, tmp, sem):
    c = lax.axis_index("c")
    s = lax.axis_index("s")
    wid = s * NC + c
    base = wid * B_PER_TILE
    tables = (e0, e1, e2, e3)

    for idx1, out in ((uidx1, usum), (iidx1, isum)):
        pltpu.sync_copy(idx1.at[pl.ds(base, B_PER_TILE)], idx)
        pltpu.async_copy(tables[0].at[idx], acc, sem).wait()
        for tbl in tables[1:]:
            pltpu.async_copy(tbl.at[idx], tmp, sem).wait()

            @pl.loop(0, B_PER_TILE)
            def _add(r):
                for q in range(4):
                    acc[r, pl.ds(q * L, L)] = (acc[r, pl.ds(q * L, L)] +
                                               tmp[r, pl.ds(q * L, L)])

        pltpu.sync_copy(acc, out.at[pl.ds(base, B_PER_TILE)])


_gather4 = pl.kernel(
    _gather_body,
    out_type=(jax.ShapeDtypeStruct((BATCH, DIM), jnp.float32),
              jax.ShapeDtypeStruct((BATCH, DIM), jnp.float32)),
    mesh=plsc.VectorSubcoreMesh(core_axis_name="c", subcore_axis_name="s",
                                num_cores=NC, num_subcores=NS),
    scratch_types=[
        pltpu.VMEM((B_PER_TILE,), jnp.int32),
        pltpu.VMEM((B_PER_TILE, DIM), jnp.float32),
        pltpu.VMEM((B_PER_TILE, DIM), jnp.float32),
        pltpu.SemaphoreType.DMA,
    ],
    compiler_params=pltpu.CompilerParams(use_tc_tiling_on_sc=False),
)


def _dot_body(u_ref, i_ref, o_ref):
    o_ref[...] = jnp.sum(u_ref[...] * i_ref[...], axis=1) * (1.0 / 16.0)


def kernel(users, items, user_w, item_w, topic_w, A_indices, A_values):
    all_emb = jnp.concatenate([user_w, item_w, topic_w], axis=0)
    row = A_indices[0]
    col = A_indices[1]
    pad = PAD_E - NNZ
    row2 = jnp.concatenate(
        [row, jnp.full((pad,), 2 ** 30, jnp.int32)]).reshape(-1, C)
    col2 = jnp.concatenate(
        [col, jnp.zeros((pad,), jnp.int32)]).reshape(-1, C)
    val2 = jnp.concatenate(
        [A_values, jnp.zeros((pad,), jnp.float32)]).reshape(-1, C)

    e0 = all_emb
    e1 = _layer(e0, row2, col2, val2)
    e2 = _layer(e1, row2, col2, val2)
    e3 = _layer(e2, row2, col2, val2)

    iidx1 = items + N_USERS
    usum, isum = _gather4(e0, e1, e2, e3, users, iidx1)

    scores = pl.pallas_call(
        _dot_body,
        out_shape=jax.ShapeDtypeStruct((BATCH,), jnp.float32),
    )(usum, isum)
    return scores


# speedup vs baseline: 10.5553x; 2.0561x over previous
import jax
import jax.numpy as jnp
from jax import lax
from jax.experimental import pallas as pl
from jax.experimental.pallas import tpu as pltpu
from jax.experimental.pallas import tpu_sc as plsc

N_USERS = 20000
N_ITEMS = 29900
N_TOPICS = 100
DIM = 64
NNZ = 800000
N_TOTAL = N_USERS + N_ITEMS + N_TOPICS
BATCH = 4096

NC, NS, L = 2, 16, 16
HD = DIM // NC
HQ = HD // L
C = 128
BLK = 6
DEPTH = 4
MACRO = 33
CHUNKS_PER_TILE = 2 * BLK * MACRO
PAD_E = NS * CHUNKS_PER_TILE * C
FLUSH = 3128
SH_ROWS = NS * FLUSH


def _process_chunk(rows, rowblk, valblk, kk, shared):
    @pl.loop(0, 8)
    def _scale(g):
        v16 = valblk[kk, pl.ds(g * L, L)]
        for k in range(16):
            e = g * 16 + k
            bv = jnp.broadcast_to(v16[k], (L,))
            rs = [rows[e, pl.ds(q * L, L)] for q in range(HQ)]
            for q in range(HQ):
                rows[e, pl.ds(q * L, L)] = rs[q] * bv

    pltpu.sync_copy(rows, shared.at[rowblk.at[kk]], add=True)


def _load_block(col2, row2, val2, cb, rb, vb, sem, goff):
    return (pltpu.async_copy(col2.at[pl.ds(goff, BLK)], cb, sem),
            pltpu.async_copy(row2.at[pl.ds(goff, BLK)], rb, sem),
            pltpu.async_copy(val2.at[pl.ds(goff, BLK)], vb, sem))


def _layer_body(src, row2, col2, val2, out, colA, rowA, valA, colB, rowB,
                valB, rows0, rows1, rows2, rows3, shared, gsem0, gsem1,
                gsem2, gsem3, isemA, isemB):
    c = lax.axis_index("c")
    s = lax.axis_index("s")
    rows_ = (rows0, rows1, rows2, rows3)
    gsem_ = (gsem0, gsem1, gsem2, gsem3)
    half = src.at[c]
    ohalf = out.at[c]

    @pl.loop(0, C)
    def _zero(r):
        for q in range(HQ):
            rows0[r, pl.ds(q * L, L)] = jnp.zeros((L,), jnp.float32)

    zbase = s * FLUSH

    @pl.loop(0, 24)
    def _zcopy(j):
        pltpu.sync_copy(rows0, shared.at[pl.ds(zbase + j * C, C)])

    pltpu.sync_copy(rows0.at[pl.ds(0, FLUSH - 24 * C)],
                    shared.at[pl.ds(zbase + 24 * C, FLUSH - 24 * C)])
    plsc.subcore_barrier()

    tile_chunk0 = s * CHUNKS_PER_TILE

    def _drain_idx(cb, rb, vb, sem):
        pltpu.make_async_copy(col2.at[pl.ds(tile_chunk0, BLK)], cb,
                              sem).wait()
        pltpu.make_async_copy(row2.at[pl.ds(tile_chunk0, BLK)], rb,
                              sem).wait()
        pltpu.make_async_copy(val2.at[pl.ds(tile_chunk0, BLK)], vb,
                              sem).wait()

    for d in _load_block(col2, row2, val2, colA, rowA, valA, isemA,
                         tile_chunk0):
        d.wait()
    for t in range(DEPTH - 1):
        pltpu.async_copy(half.at[colA.at[t]], rows_[t], gsem_[t])

    @pl.loop(0, MACRO)
    def _macro(m):
        base = tile_chunk0 + m * 2 * BLK
        dB = _load_block(col2, row2, val2, colB, rowB, valB, isemB,
                         base + BLK)
        for k in range(2 * BLK):
            par = k % DEPTH
            if k < BLK:
                rowblk, valblk, kk = rowA, valA, k
            else:
                rowblk, valblk, kk = rowB, valB, k - BLK
            if k == max(BLK - DEPTH, 0):
                for d in dB:
                    d.wait()
            if k == BLK:
                @pl.when(m < MACRO - 1)
                def _():
                    _load_block(col2, row2, val2, colA, rowA, valA,
                                isemA, base + 2 * BLK)
            npar = (k + DEPTH - 1) % DEPTH
            t = k + DEPTH - 1
            if t < BLK:
                pltpu.async_copy(half.at[colA.at[t]], rows_[npar],
                                 gsem_[npar])
            elif t < 2 * BLK:
                pltpu.async_copy(half.at[colB.at[t - BLK]], rows_[npar],
                                 gsem_[npar])
            else:
                if t == 2 * BLK:
                    @pl.when(m < MACRO - 1)
                    def _():
                        _drain_idx(colA, rowA, valA, isemA)

                @pl.when(m < MACRO - 1)
                def _():
                    pltpu.async_copy(half.at[colA.at[t - 2 * BLK]],
                                     rows_[npar], gsem_[npar])
            pltpu.make_async_copy(half.at[colA.at[0]], rows_[par],
                                  gsem_[par]).wait()
            _process_chunk(rows_[par], rowblk, valblk, kk, shared)

    plsc.subcore_barrier()

    fbase = s * FLUSH

    @pl.when(s < NS - 1)
    def _():
        pltpu.sync_copy(shared.at[pl.ds(fbase, FLUSH)],
                        ohalf.at[pl.ds(fbase, FLUSH)])

    @pl.when(s == NS - 1)
    def _():
        pltpu.sync_copy(
            shared.at[pl.ds(fbase, N_TOTAL - (NS - 1) * FLUSH)],
            ohalf.at[pl.ds(fbase, N_TOTAL - (NS - 1) * FLUSH)])


_layer = pl.kernel(
    _layer_body,
    out_type=jax.ShapeDtypeStruct((NC, N_TOTAL, HD), jnp.float32),
    mesh=plsc.VectorSubcoreMesh(core_axis_name="c", subcore_axis_name="s",
                                num_cores=NC, num_subcores=NS),
    scratch_types=[
        pltpu.VMEM((BLK, C), jnp.int32),
        pltpu.VMEM((BLK, C), jnp.int32),
        pltpu.VMEM((BLK, C), jnp.float32),
        pltpu.VMEM((BLK, C), jnp.int32),
        pltpu.VMEM((BLK, C), jnp.int32),
        pltpu.VMEM((BLK, C), jnp.float32),
        pltpu.VMEM((C, HD), jnp.float32),
        pltpu.VMEM((C, HD), jnp.float32),
        pltpu.VMEM((C, HD), jnp.float32),
        pltpu.VMEM((C, HD), jnp.float32),
        pltpu.VMEM_SHARED((SH_ROWS, HD), jnp.float32),
        pltpu.SemaphoreType.DMA,
        pltpu.SemaphoreType.DMA,
        pltpu.SemaphoreType.DMA,
        pltpu.SemaphoreType.DMA,
        pltpu.SemaphoreType.DMA,
        pltpu.SemaphoreType.DMA,
    ],
    compiler_params=pltpu.CompilerParams(use_tc_tiling_on_sc=False),
)

B_PER_TILE = BATCH // NS


def _gather_body(e0, e1, e2, e3, uidx1, iidx1, usum, isum,
                 idx, acc, tmp, sem):
    c = lax.axis_index("c")
    s = lax.axis_index("s")
    base = s * B_PER_TILE
    tables = (e0, e1, e2, e3)

    for idx1, outa in ((uidx1, usum), (iidx1, isum)):
        for h in range(B_PER_TILE // C):
            pltpu.sync_copy(idx1.at[pl.ds(base + h * C, C)], idx.at[h])
            pltpu.async_copy(tables[0].at[c].at[idx.at[h]],
                             acc.at[pl.ds(h * C, C)], sem).wait()
        for tbl in tables[1:]:
            for h in range(B_PER_TILE // C):
                pltpu.async_copy(tbl.at[c].at[idx.at[h]],
                                 tmp.at[pl.ds(h * C, C)], sem).wait()

            @pl.loop(0, B_PER_TILE)
            def _add(r):
                for q in range(HQ):
                    acc[r, pl.ds(q * L, L)] = (acc[r, pl.ds(q * L, L)] +
                                               tmp[r, pl.ds(q * L, L)])

        pltpu.sync_copy(acc, outa.at[c].at[pl.ds(base, B_PER_TILE)])


_gather4 = pl.kernel(
    _gather_body,
    out_type=(jax.ShapeDtypeStruct((NC, BATCH, HD), jnp.float32),
              jax.ShapeDtypeStruct((NC, BATCH, HD), jnp.float32)),
    mesh=plsc.VectorSubcoreMesh(core_axis_name="c", subcore_axis_name="s",
                                num_cores=NC, num_subcores=NS),
    scratch_types=[
        pltpu.VMEM((B_PER_TILE // C, C), jnp.int32),
        pltpu.VMEM((B_PER_TILE, HD), jnp.float32),
        pltpu.VMEM((B_PER_TILE, HD), jnp.float32),
        pltpu.SemaphoreType.DMA,
    ],
    compiler_params=pltpu.CompilerParams(use_tc_tiling_on_sc=False),
)


def _dot_body(u_ref, i_ref, o_ref):
    u = u_ref[...]
    i = i_ref[...]
    o_ref[...] = jnp.sum(u * i, axis=(0, 2)) * (1.0 / 16.0)


def kernel(users, items, user_w, item_w, topic_w, A_indices, A_values):
    all_emb = jnp.concatenate([user_w, item_w, topic_w], axis=0)
    e0 = jnp.stack([all_emb[:, :HD], all_emb[:, HD:]], axis=0)
    row = A_indices[0]
    col = A_indices[1]
    pad = PAD_E - NNZ
    row2 = jnp.concatenate(
        [row, jnp.zeros((pad,), jnp.int32)]).reshape(-1, C)
    col2 = jnp.concatenate(
        [col, jnp.zeros((pad,), jnp.int32)]).reshape(-1, C)
    val2 = jnp.concatenate(
        [A_values, jnp.zeros((pad,), jnp.float32)]).reshape(-1, C)

    e1 = _layer(e0, row2, col2, val2)
    e2 = _layer(e1, row2, col2, val2)
    e3 = _layer(e2, row2, col2, val2)

    iidx1 = items + N_USERS
    usum, isum = _gather4(e0, e1, e2, e3, users, iidx1)

    scores = pl.pallas_call(
        _dot_body,
        out_shape=jax.ShapeDtypeStruct((BATCH,), jnp.float32),
    )(usum, isum)
    return scores
